# baseline (speedup 1.0000x reference)
import jax
import jax.numpy as jnp
import numpy as np
from jax import lax
from jax.experimental import pallas as pl
from jax.experimental.pallas import tpu as pltpu
from jax.experimental.pallas import tpu_sc as plsc

VOCAB_SIZE = 100000
D_MODEL = 768
MAX_LEN = 4096
BATCH = 4
SEQ_LEN = 4096

NC = 2
NS = 16
NW = NC * NS
POS_PER_W = SEQ_LEN // NW
P = 32
N_CHUNK = POS_PER_W // P
N_TASK = N_CHUNK * BATCH
LANES = 16


def _sinusoidal_pos_encoding(max_len, d_model):
    pos = np.arange(max_len, dtype=np.float32)[:, None]
    i = np.arange(0, d_model, 2, dtype=np.float32)[None, :]
    angle = pos / np.power(10000.0, i / d_model)
    enc = np.zeros((max_len, d_model), dtype=np.float32)
    enc[:, 0::2] = np.sin(angle)
    enc[:, 1::2] = np.cos(angle)
    return enc


_POS_ENC_NP = _sinusoidal_pos_encoding(MAX_LEN, D_MODEL).reshape(-1)
_POS_ENC_DEV = None


def _embed_kernel(tab_hbm, idx_hbm, pos_hbm, out_hbm,
                  idx_v, pos0, pos1, tok0, tok1,
                  gsem0, gsem1, ssem0, ssem1, psem):
    wid = lax.axis_index("s") * NC + lax.axis_index("c")
    pos_base = wid * POS_PER_W

    toks = [tok0, tok1]
    gsems = [gsem0, gsem1]
    ssems = [ssem0, ssem1]
    poss = [pos0, pos1]

    icp = []
    for b in range(BATCH):
        icp.append(pltpu.async_copy(
            idx_hbm.at[pl.ds(b * SEQ_LEN + pos_base, POS_PER_W)],
            idx_v.at[pl.ds(b * POS_PER_W, POS_PER_W)], psem))
    for cp in icp:
        cp.wait()

    pltpu.sync_copy(pos_hbm.at[pl.ds(pos_base * D_MODEL, P * D_MODEL)], pos0)

    def start_gather(t):
        c, b = divmod(t, BATCH)
        isl = idx_v.at[pl.ds(b * POS_PER_W + c * P, P)]
        return pltpu.async_copy(tab_hbm.at[isl], toks[t % 2], gsems[t % 2])

    def add_pos(tok, posb):
        def add_row(r, carry):
            rbase = r * D_MODEL
            for j in range(D_MODEL // LANES):
                sl = pl.ds(j * LANES, LANES)
                tok[r, sl] = tok[r, sl] + posb[pl.ds(rbase + j * LANES, LANES)]
            return carry
        lax.fori_loop(0, P, add_row, 0)

    g_cp = [None] * N_TASK
    s_cp = [None] * N_TASK
    p_cp = [None] * N_CHUNK

    g_cp[0] = start_gather(0)
    for t in range(N_TASK):
        c, b = divmod(t, BATCH)
        if b == 0 and c + 1 < N_CHUNK:
            p_cp[c + 1] = pltpu.async_copy(
                pos_hbm.at[pl.ds((pos_base + (c + 1) * P) * D_MODEL,
                                 P * D_MODEL)],
                poss[(c + 1) % 2], psem)
        if t + 1 < N_TASK:
            if t >= 1:
                s_cp[t - 1].wait()
            g_cp[t + 1] = start_gather(t + 1)
        g_cp[t].wait()
        if b == 0 and c > 0:
            p_cp[c].wait()
        add_pos(toks[t % 2], poss[c % 2])
        s_cp[t] = pltpu.async_copy(
            toks[t % 2],
            out_hbm.at[pl.ds(b * SEQ_LEN + pos_base + c * P, P)],
            ssems[t % 2])
    s_cp[N_TASK - 2].wait()
    s_cp[N_TASK - 1].wait()


@jax.jit
def _embed(x, tok_table, pos_flat):
    x_flat = x.reshape(-1).astype(jnp.int32)
    mesh = plsc.VectorSubcoreMesh(core_axis_name="c", subcore_axis_name="s")
    run = pl.kernel(
        _embed_kernel,
        out_type=jax.ShapeDtypeStruct((BATCH * SEQ_LEN, D_MODEL), jnp.float32),
        mesh=mesh,
        scratch_types=[
            pltpu.VMEM((BATCH * POS_PER_W,), jnp.int32),
            pltpu.VMEM((P * D_MODEL,), jnp.float32),
            pltpu.VMEM((P * D_MODEL,), jnp.float32),
            pltpu.VMEM((P, D_MODEL), jnp.float32),
            pltpu.VMEM((P, D_MODEL), jnp.float32),
            pltpu.SemaphoreType.DMA,
            pltpu.SemaphoreType.DMA,
            pltpu.SemaphoreType.DMA,
            pltpu.SemaphoreType.DMA,
            pltpu.SemaphoreType.DMA,
        ],
    )
    out = run(tok_table, x_flat, pos_flat)
    return out.reshape(BATCH, SEQ_LEN, D_MODEL)


def kernel(x, tok_table):
    global _POS_ENC_DEV
    if _POS_ENC_DEV is None:
        _POS_ENC_DEV = jnp.asarray(_POS_ENC_NP)
    return _embed(x, tok_table, _POS_ENC_DEV)


# speedup vs baseline: 2.0400x; 2.0400x over previous
import jax
import jax.numpy as jnp
import numpy as np
from jax import lax
from jax.experimental import pallas as pl
from jax.experimental.pallas import tpu as pltpu
from jax.experimental.pallas import tpu_sc as plsc

VOCAB_SIZE = 100000
D_MODEL = 768
MAX_LEN = 4096
BATCH = 4
SEQ_LEN = 4096

NC = 2
NS = 16
NW = NC * NS
POS_PER_W = SEQ_LEN // NW
P = 32
N_CHUNK = POS_PER_W // P
N_TASK = N_CHUNK * BATCH
LANES = 16


def _sinusoidal_pos_encoding(max_len, d_model):
    pos = np.arange(max_len, dtype=np.float32)[:, None]
    i = np.arange(0, d_model, 2, dtype=np.float32)[None, :]
    angle = pos / np.power(10000.0, i / d_model)
    enc = np.zeros((max_len, d_model), dtype=np.float32)
    enc[:, 0::2] = np.sin(angle)
    enc[:, 1::2] = np.cos(angle)
    return enc


_POS_ENC_NP = _sinusoidal_pos_encoding(MAX_LEN, D_MODEL)
_POS_ENC_DEV = None


def _embed_kernel(tab_hbm, idx_hbm, pos_hbm, out_hbm,
                  idx_v, pos0, pos1, tok0, tok1,
                  gsem0, gsem1, ssem0, ssem1, psem):
    wid = lax.axis_index("s") * NC + lax.axis_index("c")
    pos_base = wid * POS_PER_W

    toks = [tok0, tok1]
    gsems = [gsem0, gsem1]
    ssems = [ssem0, ssem1]
    poss = [pos0, pos1]

    icp = []
    for b in range(BATCH):
        icp.append(pltpu.async_copy(
            idx_hbm.at[pl.ds(b * SEQ_LEN + pos_base, POS_PER_W)],
            idx_v.at[pl.ds(b * POS_PER_W, POS_PER_W)], psem))
    for cp in icp:
        cp.wait()

    pltpu.sync_copy(pos_hbm.at[pl.ds(pos_base, P)], pos0)

    def start_gather(t):
        c, b = divmod(t, BATCH)
        isl = idx_v.at[pl.ds(b * POS_PER_W + c * P, P)]
        return pltpu.async_copy(tab_hbm.at[isl], toks[t % 2], gsems[t % 2])

    def add_pos(tok, posb):
        def add_row(r, carry):
            for j in range(D_MODEL // LANES):
                sl = pl.ds(j * LANES, LANES)
                tok[r, sl] = tok[r, sl] + posb[r, sl]
            return carry
        lax.fori_loop(0, P, add_row, 0)

    g_cp = [None] * N_TASK
    s_cp = [None] * N_TASK
    p_cp = [None] * N_CHUNK

    g_cp[0] = start_gather(0)
    for t in range(N_TASK):
        c, b = divmod(t, BATCH)
        if b == 0 and c + 1 < N_CHUNK:
            p_cp[c + 1] = pltpu.async_copy(
                pos_hbm.at[pl.ds(pos_base + (c + 1) * P, P)],
                poss[(c + 1) % 2], psem)
        if t + 1 < N_TASK:
            if t >= 1:
                s_cp[t - 1].wait()
            g_cp[t + 1] = start_gather(t + 1)
        g_cp[t].wait()
        if b == 0 and c > 0:
            p_cp[c].wait()
        add_pos(toks[t % 2], poss[c % 2])
        s_cp[t] = pltpu.async_copy(
            toks[t % 2],
            out_hbm.at[pl.ds(b * SEQ_LEN + pos_base + c * P, P)],
            ssems[t % 2])
    s_cp[N_TASK - 2].wait()
    s_cp[N_TASK - 1].wait()


@jax.jit
def _embed(x, tok_table, pos_flat):
    x_flat = x.reshape(-1).astype(jnp.int32)
    mesh = plsc.VectorSubcoreMesh(core_axis_name="c", subcore_axis_name="s")
    run = pl.kernel(
        _embed_kernel,
        out_type=jax.ShapeDtypeStruct((BATCH * SEQ_LEN, D_MODEL), jnp.float32),
        mesh=mesh,
        scratch_types=[
            pltpu.VMEM((BATCH * POS_PER_W,), jnp.int32),
            pltpu.VMEM((P, D_MODEL), jnp.float32),
            pltpu.VMEM((P, D_MODEL), jnp.float32),
            pltpu.VMEM((P, D_MODEL), jnp.float32),
            pltpu.VMEM((P, D_MODEL), jnp.float32),
            pltpu.SemaphoreType.DMA,
            pltpu.SemaphoreType.DMA,
            pltpu.SemaphoreType.DMA,
            pltpu.SemaphoreType.DMA,
            pltpu.SemaphoreType.DMA,
        ],
    )
    out = run(tok_table, x_flat, pos_flat)
    return out.reshape(BATCH, SEQ_LEN, D_MODEL)


def kernel(x, tok_table):
    global _POS_ENC_DEV
    if _POS_ENC_DEV is None:
        _POS_ENC_DEV = jnp.asarray(_POS_ENC_NP)
    return _embed(x, tok_table, _POS_ENC_DEV)


# speedup vs baseline: 2.0786x; 1.0189x over previous
import jax
import jax.numpy as jnp
import numpy as np
from jax import lax
from jax.experimental import pallas as pl
from jax.experimental.pallas import tpu as pltpu
from jax.experimental.pallas import tpu_sc as plsc

VOCAB_SIZE = 100000
D_MODEL = 768
MAX_LEN = 4096
BATCH = 4
SEQ_LEN = 4096

NC = 2
NS = 16
NW = NC * NS
POS_PER_W = SEQ_LEN // NW
P = 32
N_CHUNK = POS_PER_W // P
N_TASK = N_CHUNK * BATCH
LANES = 16


def _sinusoidal_pos_encoding(max_len, d_model):
    pos = np.arange(max_len, dtype=np.float32)[:, None]
    i = np.arange(0, d_model, 2, dtype=np.float32)[None, :]
    angle = pos / np.power(10000.0, i / d_model)
    enc = np.zeros((max_len, d_model), dtype=np.float32)
    enc[:, 0::2] = np.sin(angle)
    enc[:, 1::2] = np.cos(angle)
    return enc


_POS_ENC_NP = _sinusoidal_pos_encoding(MAX_LEN, D_MODEL)
_POS_ENC_DEV = None


def _embed_kernel(tab_hbm, idx_hbm, pos_hbm, out_hbm,
                  idx_v, pos0, pos1, tok0, tok1,
                  gsem0, gsem1, ssem0, ssem1, psem):
    wid = lax.axis_index("s") * NC + lax.axis_index("c")
    pos_base = wid * POS_PER_W

    toks = [tok0, tok1]
    gsems = [gsem0, gsem1]
    ssems = [ssem0, ssem1]
    poss = [pos0, pos1]

    icp = []
    for b in range(BATCH):
        icp.append(pltpu.async_copy(
            idx_hbm.at[b, pl.ds(pos_base, POS_PER_W)],
            idx_v.at[pl.ds(b * POS_PER_W, POS_PER_W)], psem))
    for cp in icp:
        cp.wait()

    pltpu.sync_copy(pos_hbm.at[pl.ds(pos_base, P)], pos0)

    def start_gather(t):
        c, b = divmod(t, BATCH)
        isl = idx_v.at[pl.ds(b * POS_PER_W + c * P, P)]
        return pltpu.async_copy(tab_hbm.at[isl], toks[t % 2], gsems[t % 2])

    def add_pos(tok, posb):
        def add_row(r, carry):
            for j in range(D_MODEL // LANES):
                sl = pl.ds(j * LANES, LANES)
                tok[r, sl] = tok[r, sl] + posb[r, sl]
            return carry
        lax.fori_loop(0, P, add_row, 0)

    g_cp = [None] * N_TASK
    s_cp = [None] * N_TASK
    p_cp = [None] * N_CHUNK

    g_cp[0] = start_gather(0)
    for t in range(N_TASK):
        c, b = divmod(t, BATCH)
        if b == 0 and c + 1 < N_CHUNK:
            p_cp[c + 1] = pltpu.async_copy(
                pos_hbm.at[pl.ds(pos_base + (c + 1) * P, P)],
                poss[(c + 1) % 2], psem)
        if t + 1 < N_TASK:
            if t >= 1:
                s_cp[t - 1].wait()
            g_cp[t + 1] = start_gather(t + 1)
        g_cp[t].wait()
        if b == 0 and c > 0:
            p_cp[c].wait()
        add_pos(toks[t % 2], poss[c % 2])
        s_cp[t] = pltpu.async_copy(
            toks[t % 2],
            out_hbm.at[pl.ds(b * SEQ_LEN + pos_base + c * P, P)],
            ssems[t % 2])
    s_cp[N_TASK - 2].wait()
    s_cp[N_TASK - 1].wait()


@jax.jit
def _embed(x, tok_table, pos_flat):
    x_i32 = x.astype(jnp.int32)
    mesh = plsc.VectorSubcoreMesh(core_axis_name="c", subcore_axis_name="s")
    run = pl.kernel(
        _embed_kernel,
        out_type=jax.ShapeDtypeStruct((BATCH * SEQ_LEN, D_MODEL), jnp.float32),
        mesh=mesh,
        scratch_types=[
            pltpu.VMEM((BATCH * POS_PER_W,), jnp.int32),
            pltpu.VMEM((P, D_MODEL), jnp.float32),
            pltpu.VMEM((P, D_MODEL), jnp.float32),
            pltpu.VMEM((P, D_MODEL), jnp.float32),
            pltpu.VMEM((P, D_MODEL), jnp.float32),
            pltpu.SemaphoreType.DMA,
            pltpu.SemaphoreType.DMA,
            pltpu.SemaphoreType.DMA,
            pltpu.SemaphoreType.DMA,
            pltpu.SemaphoreType.DMA,
        ],
    )
    out = run(tok_table, x_i32, pos_flat)
    return out.reshape(BATCH, SEQ_LEN, D_MODEL)


def kernel(x, tok_table):
    global _POS_ENC_DEV
    if _POS_ENC_DEV is None:
        _POS_ENC_DEV = jnp.asarray(_POS_ENC_NP)
    return _embed(x, tok_table, _POS_ENC_DEV)
